# baseline (speedup 1.0000x reference)
import functools

import jax
import jax.numpy as jnp
import numpy as np
from jax import lax
from jax.experimental import pallas as pl
from jax.experimental.pallas import tpu as pltpu
from jax.experimental.pallas import tpu_sc as plsc

D_MODEL = 768
MAX_POSITION = 2048
NUM_CORES = 2
NUM_SUBCORES = 16
NUM_WORKERS = NUM_CORES * NUM_SUBCORES
CHUNK = 64


def _positional_encoding(length, depth_full):
    depth = depth_full // 2
    positions = jnp.arange(0, length, dtype=jnp.float32)[:, None]
    depths = jnp.arange(depth, dtype=jnp.float32)[None, :] / depth
    angle_rates = 1.0 / (10000.0 ** depths)
    angle_rads = positions * angle_rates
    return jnp.concatenate([jnp.sin(angle_rads), jnp.cos(angle_rads)], axis=-1)


def _sc_gather(table, idx):
    n = idx.shape[0]
    rpw = n // NUM_WORKERS
    n_chunks = rpw // CHUNK
    mesh = plsc.VectorSubcoreMesh(core_axis_name="c", subcore_axis_name="s")

    @functools.partial(
        pl.kernel,
        mesh=mesh,
        out_type=jax.ShapeDtypeStruct((n, D_MODEL), jnp.float32),
        scratch_types=[
            pltpu.VMEM((rpw,), jnp.int32),
            pltpu.VMEM((CHUNK, D_MODEL), jnp.float32),
            pltpu.VMEM((CHUNK, D_MODEL), jnp.float32),
            pltpu.SemaphoreType.DMA,
            pltpu.SemaphoreType.DMA,
        ],
    )
    def k(table_hbm, idx_hbm, out_hbm, idx_v, buf0, buf1, sem0, sem1):
        wid = lax.axis_index("s") * NUM_CORES + lax.axis_index("c")
        base = wid * rpw
        pltpu.sync_copy(idx_hbm.at[pl.ds(base, rpw)], idx_v)
        bufs = (buf0, buf1)
        sems = (sem0, sem1)
        copies = [None] * n_chunks
        copies[0] = pltpu.async_copy(
            table_hbm.at[idx_v.at[pl.ds(0, CHUNK)]], bufs[0], sems[0]
        )
        for c in range(n_chunks):
            copies[c].wait()
            if c + 1 < n_chunks:
                copies[c + 1] = pltpu.async_copy(
                    table_hbm.at[idx_v.at[pl.ds((c + 1) * CHUNK, CHUNK)]],
                    bufs[(c + 1) % 2],
                    sems[(c + 1) % 2],
                )
            pltpu.sync_copy(bufs[c % 2], out_hbm.at[pl.ds(base + c * CHUNK, CHUNK)])

    return k(table, idx)


def _tc_fixup(gathered, pos, batch, seq_len, scale):
    block = 256
    n_seq_blocks = seq_len // block

    def body(g_ref, p_ref, o_ref):
        o_ref[...] = g_ref[...] * scale + p_ref[...]

    return pl.pallas_call(
        body,
        out_shape=jax.ShapeDtypeStruct((batch * seq_len, D_MODEL), jnp.float32),
        grid=(n_seq_blocks, batch),
        in_specs=[
            pl.BlockSpec(
                (block, D_MODEL), lambda i, b: (b * n_seq_blocks + i, 0)
            ),
            pl.BlockSpec((block, D_MODEL), lambda i, b: (i, 0)),
        ],
        out_specs=pl.BlockSpec(
            (block, D_MODEL), lambda i, b: (b * n_seq_blocks + i, 0)
        ),
    )(gathered, pos)


def kernel(inputs, table):
    batch, seq_len = inputs.shape
    idx = inputs.astype(jnp.int32)
    pos = _positional_encoding(MAX_POSITION, D_MODEL)[:seq_len]
    scale = float(np.sqrt(np.float32(D_MODEL)))
    half = batch // 2
    outs = []
    for h in range(2):
        idx_h = jnp.reshape(idx[h * half : (h + 1) * half], (half * seq_len,))
        gathered = _sc_gather(table, idx_h)
        outs.append(_tc_fixup(gathered, pos, half, seq_len, scale))
    out = jnp.concatenate(outs, axis=0)
    return jnp.reshape(out, (batch, seq_len, D_MODEL))


# speedup vs baseline: 1.2700x; 1.2700x over previous
import functools

import jax
import jax.numpy as jnp
import numpy as np
from jax import lax
from jax.experimental import pallas as pl
from jax.experimental.pallas import tpu as pltpu
from jax.experimental.pallas import tpu_sc as plsc

D_MODEL = 768
MAX_POSITION = 2048
NUM_CORES = 2
NUM_SUBCORES = 16
NUM_WORKERS = NUM_CORES * NUM_SUBCORES
CHUNK = 64


def _positional_encoding(length, depth_full):
    depth = depth_full // 2
    positions = np.arange(0, length, dtype=np.float32)[:, None]
    depths = np.arange(depth, dtype=np.float32)[None, :] / np.float32(depth)
    angle_rates = (1.0 / (10000.0 ** depths)).astype(np.float32)
    angle_rads = positions * angle_rates
    enc = np.concatenate([np.sin(angle_rads), np.cos(angle_rads)], axis=-1)
    return jnp.asarray(enc.astype(np.float32))


def _sc_gather(table, idx):
    n = idx.shape[0]
    rpw = n // NUM_WORKERS
    n_chunks = rpw // CHUNK
    mesh = plsc.VectorSubcoreMesh(core_axis_name="c", subcore_axis_name="s")

    @functools.partial(
        pl.kernel,
        mesh=mesh,
        out_type=jax.ShapeDtypeStruct((n, D_MODEL), jnp.float32),
        scratch_types=[
            pltpu.VMEM((rpw,), jnp.int32),
            pltpu.VMEM((CHUNK, D_MODEL), jnp.float32),
            pltpu.VMEM((CHUNK, D_MODEL), jnp.float32),
            pltpu.SemaphoreType.DMA,
            pltpu.SemaphoreType.DMA,
        ],
    )
    def k(table_hbm, idx_hbm, out_hbm, idx_v, buf0, buf1, sem0, sem1):
        wid = lax.axis_index("s") * NUM_CORES + lax.axis_index("c")
        base = wid * rpw
        pltpu.sync_copy(idx_hbm.at[pl.ds(base, rpw)], idx_v)
        bufs = (buf0, buf1)
        sems = (sem0, sem1)
        copies = [None] * n_chunks
        copies[0] = pltpu.async_copy(
            table_hbm.at[idx_v.at[pl.ds(0, CHUNK)]], bufs[0], sems[0]
        )
        for c in range(n_chunks):
            copies[c].wait()
            if c + 1 < n_chunks:
                copies[c + 1] = pltpu.async_copy(
                    table_hbm.at[idx_v.at[pl.ds((c + 1) * CHUNK, CHUNK)]],
                    bufs[(c + 1) % 2],
                    sems[(c + 1) % 2],
                )
            pltpu.sync_copy(bufs[c % 2], out_hbm.at[pl.ds(base + c * CHUNK, CHUNK)])

    return k(table, idx)


def _tc_fixup(gathered, pos, batch, seq_len, scale):
    block = 256
    n_seq_blocks = seq_len // block

    def body(g_ref, p_ref, o_ref):
        o_ref[...] = g_ref[...] * scale + p_ref[...]

    return pl.pallas_call(
        body,
        out_shape=jax.ShapeDtypeStruct((batch * seq_len, D_MODEL), jnp.float32),
        grid=(n_seq_blocks, batch),
        in_specs=[
            pl.BlockSpec(
                (block, D_MODEL), lambda i, b: (b * n_seq_blocks + i, 0)
            ),
            pl.BlockSpec((block, D_MODEL), lambda i, b: (i, 0)),
        ],
        out_specs=pl.BlockSpec(
            (block, D_MODEL), lambda i, b: (b * n_seq_blocks + i, 0)
        ),
    )(gathered, pos)


def kernel(inputs, table):
    batch, seq_len = inputs.shape
    idx = inputs.astype(jnp.int32)
    pos = _positional_encoding(MAX_POSITION, D_MODEL)[:seq_len]
    scale = float(np.sqrt(np.float32(D_MODEL)))
    idx_flat = jnp.reshape(idx, (batch * seq_len,))
    gathered = _sc_gather(table, idx_flat)
    out = _tc_fixup(gathered, pos, batch, seq_len, scale)
    return jnp.reshape(out, (batch, seq_len, D_MODEL))


# speedup vs baseline: 1.7455x; 1.3744x over previous
import functools

import jax
import jax.numpy as jnp
import numpy as np
from jax import lax
from jax.experimental import pallas as pl
from jax.experimental.pallas import tpu as pltpu
from jax.experimental.pallas import tpu_sc as plsc

D_MODEL = 768
MAX_POSITION = 2048
LANES = 16
NUM_CORES = 2
NUM_SUBCORES = 16
NUM_WORKERS = NUM_CORES * NUM_SUBCORES
CHUNK = 32
NBUF = 4
SCALE = float(np.sqrt(np.float32(D_MODEL)))


def _positional_encoding(length, depth_full):
    depth = depth_full // 2
    positions = np.arange(0, length, dtype=np.float32)[:, None]
    depths = np.arange(depth, dtype=np.float32)[None, :] / np.float32(depth)
    angle_rates = (1.0 / (10000.0 ** depths)).astype(np.float32)
    angle_rads = positions * angle_rates
    enc = np.concatenate([np.sin(angle_rads), np.cos(angle_rads)], axis=-1)
    return jnp.asarray(enc.astype(np.float32))


def _sc_embed(table, idx, pos, batch, seq_len):
    n_rows = batch * seq_len
    ppw = seq_len // NUM_WORKERS
    pos_chunks = ppw // CHUNK
    n_chunks = pos_chunks * batch
    mesh = plsc.VectorSubcoreMesh(core_axis_name="c", subcore_axis_name="s")

    buf_types = [pltpu.VMEM((CHUNK, D_MODEL), jnp.float32)] * NBUF
    gsem_types = [pltpu.SemaphoreType.DMA] * NBUF
    wsem_types = [pltpu.SemaphoreType.DMA] * NBUF

    @functools.partial(
        pl.kernel,
        mesh=mesh,
        out_type=jax.ShapeDtypeStruct((n_rows, D_MODEL), jnp.float32),
        scratch_types=[
            pltpu.VMEM((batch * ppw,), jnp.int32),
            pltpu.VMEM((CHUNK, D_MODEL), jnp.float32),
        ]
        + buf_types
        + gsem_types
        + wsem_types,
    )
    def k(table_hbm, idx_hbm, pos_hbm, out_hbm, idx_v, pos_v, *rest):
        bufs = rest[:NBUF]
        gsems = rest[NBUF : 2 * NBUF]
        wsems = rest[2 * NBUF :]
        wid = lax.axis_index("s") * NUM_CORES + lax.axis_index("c")
        pbase = wid * ppw

        for b in range(batch):
            pltpu.sync_copy(
                idx_hbm.at[pl.ds(b * seq_len + pbase, ppw)],
                idx_v.at[pl.ds(b * ppw, ppw)],
            )

        def chunk_pb(c):
            return c // batch, c % batch

        def gather(c):
            p, b = chunk_pb(c)
            return pltpu.async_copy(
                table_hbm.at[idx_v.at[pl.ds(b * ppw + p * CHUNK, CHUNK)]],
                bufs[c % NBUF],
                gsems[c % NBUF],
            )

        LOOKAHEAD = 2
        gcopies = [None] * n_chunks
        wcopies = [None] * n_chunks
        for c in range(min(LOOKAHEAD, n_chunks)):
            gcopies[c] = gather(c)

        for c in range(n_chunks):
            nxt = c + LOOKAHEAD
            if nxt < n_chunks:
                prev_occupant = nxt - NBUF
                if prev_occupant >= 0:
                    wcopies[prev_occupant].wait()
                gcopies[nxt] = gather(nxt)

            p, b = chunk_pb(c)
            if b == 0:
                pltpu.sync_copy(pos_hbm.at[pl.ds(pbase + p * CHUNK, CHUNK)], pos_v)
            gcopies[c].wait()
            buf = bufs[c % NBUF]

            @pl.loop(0, CHUNK)
            def _row(r):
                for j in range(0, D_MODEL, LANES):
                    buf[r, pl.ds(j, LANES)] = (
                        buf[r, pl.ds(j, LANES)] * SCALE
                        + pos_v[r, pl.ds(j, LANES)]
                    )

            wcopies[c] = pltpu.async_copy(
                buf,
                out_hbm.at[pl.ds(b * seq_len + pbase + p * CHUNK, CHUNK)],
                wsems[c % NBUF],
            )

        for c in range(max(0, n_chunks - NBUF), n_chunks):
            wcopies[c].wait()

    return k(table, idx, pos)


def kernel(inputs, table):
    batch, seq_len = inputs.shape
    idx = jnp.reshape(inputs.astype(jnp.int32), (batch * seq_len,))
    pos = _positional_encoding(MAX_POSITION, D_MODEL)[:seq_len]
    out = _sc_embed(table, idx, pos, batch, seq_len)
    return jnp.reshape(out, (batch, seq_len, D_MODEL))
